# baseline (speedup 1.0000x reference)
import functools

import jax
import jax.numpy as jnp
from jax import lax
from jax.experimental import pallas as pl
from jax.experimental.pallas import tpu as pltpu
from jax.experimental.pallas import tpu_sc as plsc

_CHUNK_ROWS = 64


@functools.cache
def _sc_copy(b, s, d, dtype):
    info = plsc.get_sparse_core_info()
    nw = info.num_cores * info.num_subcores
    rows_per_w = s // nw
    n_chunks = rows_per_w // _CHUNK_ROWS
    mesh = plsc.VectorSubcoreMesh(core_axis_name="c", subcore_axis_name="s")

    @functools.partial(
        pl.kernel,
        mesh=mesh,
        out_type=jax.ShapeDtypeStruct((b, s, d), dtype),
        scratch_types=[pltpu.VMEM((_CHUNK_ROWS, d), dtype)],
    )
    def k(emb_hbm, out_hbm, buf):
        wid = lax.axis_index("s") * info.num_cores + lax.axis_index("c")
        base = wid * rows_per_w
        for c in range(n_chunks):
            off = base + c * _CHUNK_ROWS
            pltpu.sync_copy(emb_hbm.at[pl.ds(off, _CHUNK_ROWS), :], buf)
            for bi in range(b):
                pltpu.sync_copy(buf, out_hbm.at[bi, pl.ds(off, _CHUNK_ROWS), :])

    return k


def kernel(x, emb):
    b, s, d = x.shape
    return _sc_copy(b, s, d, emb.dtype)(emb)


# speedup vs baseline: 1.0119x; 1.0119x over previous
import functools

import jax
import jax.numpy as jnp
from jax import lax
from jax.experimental import pallas as pl
from jax.experimental.pallas import tpu as pltpu
from jax.experimental.pallas import tpu_sc as plsc

_CHUNK_ROWS = 32
_NBUF = 3


@functools.cache
def _sc_copy(b, s, d, dtype):
    info = plsc.get_sparse_core_info()
    nw = info.num_cores * info.num_subcores
    rows_per_w = s // nw
    n_chunks = rows_per_w // _CHUNK_ROWS
    mesh = plsc.VectorSubcoreMesh(core_axis_name="c", subcore_axis_name="s")

    @functools.partial(
        pl.kernel,
        mesh=mesh,
        out_type=jax.ShapeDtypeStruct((b, s, d), dtype),
        scratch_types=[
            pltpu.VMEM((_NBUF, _CHUNK_ROWS, d), dtype),
            pltpu.SemaphoreType.DMA,
            pltpu.SemaphoreType.DMA,
        ],
    )
    def k(emb_hbm, out_hbm, buf, rsem, wsem):
        wid = lax.axis_index("s") * info.num_cores + lax.axis_index("c")
        base = wid * rows_per_w

        def rd(c):
            off = base + c * _CHUNK_ROWS
            return pltpu.async_copy(
                emb_hbm.at[pl.ds(off, _CHUNK_ROWS), :], buf.at[c % _NBUF], rsem
            )

        def wr(c):
            off = base + c * _CHUNK_ROWS
            return [
                pltpu.async_copy(
                    buf.at[c % _NBUF], out_hbm.at[bi, pl.ds(off, _CHUNK_ROWS), :], wsem
                )
                for bi in range(b)
            ]

        reads = {}
        writes = {}
        for c in range(min(2, n_chunks)):
            reads[c] = rd(c)
        for c in range(n_chunks):
            reads[c].wait()
            writes[c] = wr(c)
            n = c + 2
            if n < n_chunks:
                prev = n - _NBUF
                if prev >= 0:
                    for w in writes[prev]:
                        w.wait()
                    del writes[prev]
                reads[n] = rd(n)
        for c in sorted(writes):
            for w in writes[c]:
                w.wait()

    return k


def kernel(x, emb):
    b, s, d = x.shape
    return _sc_copy(b, s, d, emb.dtype)(emb)
